# baseline (speedup 1.0000x reference)
import functools

import jax
import jax.numpy as jnp
from jax import lax
from jax.experimental import pallas as pl
from jax.experimental.pallas import tpu as pltpu
from jax.experimental.pallas import tpu_sc as plsc

EPS = 1e-5
K = 20
N = 1024
B = 8
NEG = -3e38
F32 = jnp.float32
HIGH = jax.lax.Precision.HIGHEST
NTILE = 16
NCORE = 2
RPT = N // NTILE
BPC = B // NCORE


def _graph_body(x_ref, xf_ref, w1_ref, idx_ref, dinv_ref, sf_ref, hd_ref):
    xr = x_ref[0]
    xc = xf_ref[0]
    sq = xr * xr
    xx_r = (sq[0:1, :] + sq[1:2, :]) + sq[2:3, :]
    sqc = xc * xc
    xx_c = (sqc[:, 0:1] + sqc[:, 1:2]) + sqc[:, 2:3]
    inner = -2.0 * lax.dot_general(xc, xr, (((1,), (0,)), ((), ())),
                                   preferred_element_type=F32)
    work = (-xx_r - inner) - xx_c
    iota_s = lax.broadcasted_iota(jnp.int32, (N, N), 0)
    for r in range(K):
        m = jnp.max(work, axis=0, keepdims=True)
        am = jnp.min(jnp.where(work == m, iota_s, N), axis=0, keepdims=True)
        idx_ref[0, r:r + 1, :] = am
        work = jnp.where(iota_s == am, NEG, work)
    selt = work <= NEG * 0.5
    diag = iota_s == lax.broadcasted_iota(jnp.int32, (N, N), 1)
    awt = jnp.where(diag | selt, 1.0, 0.0).astype(F32)
    deg = jnp.sum(awt, axis=1, keepdims=True)
    dinv = jnp.where(deg > 0, 1.0 / jnp.sqrt(deg), 0.0)
    sf = jnp.sum(jnp.where(diag & selt, 1.0, 0.0), axis=1, keepdims=True)
    dinv_ref[0] = dinv
    sf_ref[0] = sf
    hd_ref[0] = dinv * lax.dot_general(xc, w1_ref[...], (((1,), (0,)), ((), ())),
                                       preferred_element_type=F32)


@functools.lru_cache(maxsize=None)
def _make_sc_agg(d):
    nslice = d // 64
    ntask = B * nslice
    nworker = NCORE * NTILE
    rounds = -(-ntask // nworker)
    half = (N // 2) * 64
    mesh = plsc.VectorSubcoreMesh(core_axis_name="c", subcore_axis_name="s")

    @functools.partial(
        pl.kernel,
        mesh=mesh,
        compiler_params=pltpu.CompilerParams(needs_layout_passes=False),
        out_type=jax.ShapeDtypeStruct((B, nslice, 1, N * 64), F32),
        scratch_types=[
            pltpu.VMEM((K * N,), jnp.int32),
            pltpu.VMEM((half,), F32),
            pltpu.VMEM((N * 64,), F32),
        ],
    )
    def sck(idx_hbm, hd_hbm, zero_hbm, z_hbm, idx_v, sbuf, acc):
        c = lax.axis_index("c")
        s = lax.axis_index("s")
        w = s * NCORE + c
        iota = lax.iota(jnp.int32, 16)
        izero = iota * 0
        for t in range(rounds):
            task = w + t * nworker

            @pl.when(task < ntask)
            def _():
                bg = task // nslice
                ci = task % nslice
                pltpu.sync_copy(zero_hbm, acc)
                pltpu.sync_copy(idx_hbm.at[bg, 0], idx_v)
                for hf in range(2):
                    pltpu.sync_copy(hd_hbm.at[bg, ci, 0, pl.ds(hf * half, half)],
                                    sbuf)

                    def ebody(q, _):
                        k = q // 32
                        r0 = (q % 32) * 16
                        srcl = r0 + iota
                        dstv = plsc.load_gather(
                            idx_v, [k * N + hf * (N // 2) + srcl])
                        dst64 = dstv * 64
                        src64 = srcl * 64
                        for f in range(64):
                            v = plsc.load_gather(sbuf, [src64 + f])
                            plsc.addupdate_scatter(acc, [dst64 + f], v)
                        return 0

                    lax.fori_loop(0, K * 32, ebody, 0)
                pltpu.sync_copy(acc, z_hbm.at[bg, ci, 0])

    return sck


def _sc_aggregate(d):
    sck = _make_sc_agg(d)

    def run(idxt, hd, zeros64):
        nslice = d // 64
        hd4 = hd.reshape(B, N, nslice, 64).transpose(0, 2, 1, 3)
        z4 = sck(idxt.reshape(B, 1, K * N), hd4.reshape(B, nslice, 1, N * 64),
                 zeros64)
        z4 = z4.reshape(B, nslice, N, 64)
        return z4.transpose(0, 2, 1, 3).reshape(B, N, d)

    return run


def _epilogue(z_ref, hd_ref, dinv_ref, sf_ref, b_ref, g_ref, be_ref):
    dinv = dinv_ref[0]
    sf = sf_ref[0]
    inv_sqrt = 1.0 / jnp.sqrt(jnp.array(1.0 + EPS, F32))
    t = dinv * (z_ref[0] + (1.0 - sf) * hd_ref[0]) + b_ref[...]
    t = t * (g_ref[...] * inv_sqrt) + be_ref[...]
    return jnp.where(t >= 0, t, 0.2 * t)


def _layer_body(z_ref, hd_ref, dinv_ref, sf_ref, b_ref, g_ref, be_ref,
                w_ref, out_ref):
    h = _epilogue(z_ref, hd_ref, dinv_ref, sf_ref, b_ref, g_ref, be_ref)
    out_ref[0] = dinv_ref[0] * lax.dot_general(
        h, w_ref[...], (((1,), (0,)), ((), ())), preferred_element_type=F32)


def _pool_body(z_ref, hd_ref, dinv_ref, sf_ref, b_ref, g_ref, be_ref,
               out_ref):
    h = _epilogue(z_ref, hd_ref, dinv_ref, sf_ref, b_ref, g_ref, be_ref)
    out_ref[0] = jnp.sum(h, axis=0, keepdims=True)


def _head_body(s_ref, l1_ref, g6_ref, e6_ref, l2_ref, l2b_ref,
               g7_ref, e7_ref, l3_ref, l3b_ref, out_ref):
    s = s_ref[...]
    inv_sqrt = 1.0 / jnp.sqrt(jnp.array(1.0 + EPS, F32))
    w1a = l1_ref[0:N, :]
    w1b = l1_ref[N:2 * N, :]
    y = (lax.dot_general(s * (1.0 / N), w1a, (((1,), (0,)), ((), ())),
                         preferred_element_type=F32)
         + lax.dot_general(s, w1b, (((1,), (0,)), ((), ())),
                           preferred_element_type=F32))
    y = y * (g6_ref[...] * inv_sqrt) + e6_ref[...]
    y = jnp.where(y >= 0, y, 0.2 * y)
    y = lax.dot_general(y, l2_ref[...], (((1,), (0,)), ((), ())),
                        preferred_element_type=F32)
    y = (y + l2b_ref[...]) * (g7_ref[...] * inv_sqrt) + e7_ref[...]
    y = jnp.where(y >= 0, y, 0.2 * y)
    out_ref[...] = (lax.dot_general(y, l3_ref[...], (((1,), (0,)), ((), ())),
                                    preferred_element_type=F32)
                    + l3b_ref[...])


def _full(shape):
    return pl.BlockSpec(shape, lambda b: (0,) * len(shape))


def _bspec(shape):
    return pl.BlockSpec((1,) + shape, lambda b: (b,) + (0,) * len(shape))


def kernel(x, W1, b1, W2, b2, W3, b3, W4, b4, W5, b5,
           g1, be1, g2, be2, g3, be3, g4, be4, g5, be5, g6, be6, g7, be7,
           L1W, L2W, L2b, L3W, L3b):
    xf = jnp.transpose(x, (0, 2, 1))
    vec = lambda v: v.reshape(1, -1)

    idxt, dinv, sf, hd = pl.pallas_call(
        _graph_body,
        grid=(B,),
        in_specs=[_bspec((3, N)), _bspec((N, 3)), _full(W1.shape)],
        out_specs=[_bspec((K, N)), _bspec((N, 1)), _bspec((N, 1)),
                   _bspec((N, W1.shape[1]))],
        out_shape=[jax.ShapeDtypeStruct((B, K, N), jnp.int32),
                   jax.ShapeDtypeStruct((B, N, 1), F32),
                   jax.ShapeDtypeStruct((B, N, 1), F32),
                   jax.ShapeDtypeStruct((B, N, W1.shape[1]), F32)],
    )(x, xf, W1)

    layer_params = ((b1, g1, be1, W2), (b2, g2, be2, W3),
                    (b3, g3, be3, W4), (b4, g4, be4, W5),
                    (b5, g5, be5, None))
    zeros64 = jnp.zeros((N * 64,), F32)
    for bb, g, be, wn in layer_params:
        d = hd.shape[-1]
        z = _sc_aggregate(d)(idxt, hd, zeros64)
        common = (z, hd, dinv, sf, vec(bb), vec(g), vec(be))
        common_specs = [_bspec((N, d)), _bspec((N, d)), _bspec((N, 1)),
                        _bspec((N, 1)), _full((1, d)), _full((1, d)),
                        _full((1, d))]
        if wn is not None:
            hd = pl.pallas_call(
                _layer_body,
                grid=(B,),
                in_specs=common_specs + [_full(wn.shape)],
                out_specs=_bspec((N, wn.shape[1])),
                out_shape=jax.ShapeDtypeStruct((B, N, wn.shape[1]), F32),
            )(*common, wn)
        else:
            s = pl.pallas_call(
                _pool_body,
                grid=(B,),
                in_specs=common_specs,
                out_specs=pl.BlockSpec((1, 1, N), lambda b: (b, 0, 0)),
                out_shape=jax.ShapeDtypeStruct((B, 1, N), F32),
            )(*common)
    s = s.reshape(B, N)

    out = pl.pallas_call(
        _head_body,
        out_shape=jax.ShapeDtypeStruct((B, 40), F32),
    )(s, L1W, vec(g6), vec(be6), L2W, vec(L2b), vec(g7), vec(be7),
      L3W, vec(L3b))
    return out


# speedup vs baseline: 3.6709x; 3.6709x over previous
import functools

import jax
import jax.numpy as jnp
from jax import lax
from jax.experimental import pallas as pl
from jax.experimental.pallas import tpu as pltpu
from jax.experimental.pallas import tpu_sc as plsc

EPS = 1e-5
K = 20
N = 1024
B = 8
NEG = -3e38
F32 = jnp.float32
HIGH = jax.lax.Precision.HIGHEST
NTILE = 16
NCORE = 2
RPT = N // NTILE
BPC = B // NCORE


def _graph_body(x_ref, xf_ref, w1_ref, idx_ref, dinv_ref, sf_ref, hd_ref):
    xr = x_ref[0]
    xc = xf_ref[0]
    sq = xr * xr
    xx_r = (sq[0:1, :] + sq[1:2, :]) + sq[2:3, :]
    sqc = xc * xc
    xx_c = (sqc[:, 0:1] + sqc[:, 1:2]) + sqc[:, 2:3]
    inner = -2.0 * lax.dot_general(xc, xr, (((1,), (0,)), ((), ())),
                                   preferred_element_type=F32)
    work = (-xx_r - inner) - xx_c
    iota_s = lax.broadcasted_iota(jnp.int32, (N, N), 0)
    for r in range(K):
        m = jnp.max(work, axis=0, keepdims=True)
        am = jnp.min(jnp.where(work == m, iota_s, N), axis=0, keepdims=True)
        idx_ref[0, r:r + 1, :] = am
        work = jnp.where(iota_s == am, NEG, work)
    selt = work <= NEG * 0.5
    diag = iota_s == lax.broadcasted_iota(jnp.int32, (N, N), 1)
    awt = jnp.where(diag | selt, 1.0, 0.0).astype(F32)
    deg = jnp.sum(awt, axis=1, keepdims=True)
    dinv = jnp.where(deg > 0, 1.0 / jnp.sqrt(deg), 0.0)
    sf = jnp.sum(jnp.where(diag & selt, 1.0, 0.0), axis=1, keepdims=True)
    dinv_ref[0] = dinv
    sf_ref[0] = sf
    hd_ref[0] = dinv * lax.dot_general(xc, w1_ref[...], (((1,), (0,)), ((), ())),
                                       preferred_element_type=F32)


@functools.lru_cache(maxsize=None)
def _make_sc_agg(d):
    nslice = d // 64
    ntask = B * nslice
    nworker = NCORE * NTILE
    rounds = -(-ntask // nworker)
    half = (N // 2) * 64
    mesh = plsc.VectorSubcoreMesh(core_axis_name="c", subcore_axis_name="s")

    @functools.partial(
        pl.kernel,
        mesh=mesh,
        compiler_params=pltpu.CompilerParams(needs_layout_passes=False),
        out_type=jax.ShapeDtypeStruct((B, nslice, 1, N * 64), F32),
        scratch_types=[
            pltpu.VMEM((K * N,), jnp.int32),
            pltpu.VMEM((half,), F32),
            pltpu.VMEM((N * 64,), F32),
        ],
    )
    def sck(idx_hbm, hd_hbm, zero_hbm, z_hbm, idx_v, sbuf, acc):
        c = lax.axis_index("c")
        s = lax.axis_index("s")
        w = s * NCORE + c
        iota = lax.iota(jnp.int32, 16)
        for t in range(rounds):
            task = w + t * nworker

            @pl.when(task < ntask)
            def _():
                bg = task // nslice
                ci = task % nslice
                pltpu.sync_copy(zero_hbm, acc)
                pltpu.sync_copy(idx_hbm.at[bg, 0], idx_v)
                for hf in range(2):
                    pltpu.sync_copy(hd_hbm.at[bg, ci, hf, 0], sbuf)

                    def ebody(q, _):
                        k = q // 32
                        r0 = (q % 32) * 16
                        srcl = r0 + iota
                        dstv = plsc.load_gather(
                            idx_v, [k * N + hf * (N // 2) + srcl])
                        for f in range(64):
                            v = plsc.load_gather(sbuf, [f * (N // 2) + srcl])
                            plsc.addupdate_scatter(acc, [f * N + dstv], v)
                        return 0

                    lax.fori_loop(0, K * 32, ebody, 0)
                pltpu.sync_copy(acc, z_hbm.at[bg, ci, 0])

    return sck


def _sc_aggregate(d):
    sck = _make_sc_agg(d)

    def run(idxt, hd, zeros64):
        nslice = d // 64
        hd5 = hd.reshape(B, 2, N // 2, nslice, 64).transpose(0, 3, 1, 4, 2)
        hd5 = hd5.reshape(B, nslice, 2, 1, half_flat)
        z5 = sck(idxt.reshape(B, 1, K * N), hd5, zeros64)
        z = z5.reshape(B, nslice, 64, N).transpose(0, 3, 1, 2)
        return z.reshape(B, N, d)

    return run


half_flat = (N // 2) * 64


def _epilogue(z_ref, hd_ref, dinv_ref, sf_ref, b_ref, g_ref, be_ref):
    dinv = dinv_ref[0]
    sf = sf_ref[0]
    inv_sqrt = 1.0 / jnp.sqrt(jnp.array(1.0 + EPS, F32))
    t = dinv * (z_ref[0] + (1.0 - sf) * hd_ref[0]) + b_ref[...]
    t = t * (g_ref[...] * inv_sqrt) + be_ref[...]
    return jnp.where(t >= 0, t, 0.2 * t)


def _layer_body(z_ref, hd_ref, dinv_ref, sf_ref, b_ref, g_ref, be_ref,
                w_ref, out_ref):
    h = _epilogue(z_ref, hd_ref, dinv_ref, sf_ref, b_ref, g_ref, be_ref)
    out_ref[0] = dinv_ref[0] * lax.dot_general(
        h, w_ref[...], (((1,), (0,)), ((), ())), preferred_element_type=F32)


def _pool_body(z_ref, hd_ref, dinv_ref, sf_ref, b_ref, g_ref, be_ref,
               out_ref):
    h = _epilogue(z_ref, hd_ref, dinv_ref, sf_ref, b_ref, g_ref, be_ref)
    out_ref[0] = jnp.sum(h, axis=0, keepdims=True)


def _head_body(s_ref, l1_ref, g6_ref, e6_ref, l2_ref, l2b_ref,
               g7_ref, e7_ref, l3_ref, l3b_ref, out_ref):
    s = s_ref[...]
    inv_sqrt = 1.0 / jnp.sqrt(jnp.array(1.0 + EPS, F32))
    w1a = l1_ref[0:N, :]
    w1b = l1_ref[N:2 * N, :]
    y = (lax.dot_general(s * (1.0 / N), w1a, (((1,), (0,)), ((), ())),
                         preferred_element_type=F32)
         + lax.dot_general(s, w1b, (((1,), (0,)), ((), ())),
                           preferred_element_type=F32))
    y = y * (g6_ref[...] * inv_sqrt) + e6_ref[...]
    y = jnp.where(y >= 0, y, 0.2 * y)
    y = lax.dot_general(y, l2_ref[...], (((1,), (0,)), ((), ())),
                        preferred_element_type=F32)
    y = (y + l2b_ref[...]) * (g7_ref[...] * inv_sqrt) + e7_ref[...]
    y = jnp.where(y >= 0, y, 0.2 * y)
    out_ref[...] = (lax.dot_general(y, l3_ref[...], (((1,), (0,)), ((), ())),
                                    preferred_element_type=F32)
                    + l3b_ref[...])


def _full(shape):
    return pl.BlockSpec(shape, lambda b: (0,) * len(shape))


def _bspec(shape):
    return pl.BlockSpec((1,) + shape, lambda b: (b,) + (0,) * len(shape))


def kernel(x, W1, b1, W2, b2, W3, b3, W4, b4, W5, b5,
           g1, be1, g2, be2, g3, be3, g4, be4, g5, be5, g6, be6, g7, be7,
           L1W, L2W, L2b, L3W, L3b):
    xf = jnp.transpose(x, (0, 2, 1))
    vec = lambda v: v.reshape(1, -1)

    idxt, dinv, sf, hd = pl.pallas_call(
        _graph_body,
        grid=(B,),
        in_specs=[_bspec((3, N)), _bspec((N, 3)), _full(W1.shape)],
        out_specs=[_bspec((K, N)), _bspec((N, 1)), _bspec((N, 1)),
                   _bspec((N, W1.shape[1]))],
        out_shape=[jax.ShapeDtypeStruct((B, K, N), jnp.int32),
                   jax.ShapeDtypeStruct((B, N, 1), F32),
                   jax.ShapeDtypeStruct((B, N, 1), F32),
                   jax.ShapeDtypeStruct((B, N, W1.shape[1]), F32)],
    )(x, xf, W1)

    layer_params = ((b1, g1, be1, W2), (b2, g2, be2, W3),
                    (b3, g3, be3, W4), (b4, g4, be4, W5),
                    (b5, g5, be5, None))
    zeros64 = jnp.zeros((N * 64,), F32)
    for bb, g, be, wn in layer_params:
        d = hd.shape[-1]
        z = _sc_aggregate(d)(idxt, hd, zeros64)
        common = (z, hd, dinv, sf, vec(bb), vec(g), vec(be))
        common_specs = [_bspec((N, d)), _bspec((N, d)), _bspec((N, 1)),
                        _bspec((N, 1)), _full((1, d)), _full((1, d)),
                        _full((1, d))]
        if wn is not None:
            hd = pl.pallas_call(
                _layer_body,
                grid=(B,),
                in_specs=common_specs + [_full(wn.shape)],
                out_specs=_bspec((N, wn.shape[1])),
                out_shape=jax.ShapeDtypeStruct((B, N, wn.shape[1]), F32),
            )(*common, wn)
        else:
            s = pl.pallas_call(
                _pool_body,
                grid=(B,),
                in_specs=common_specs,
                out_specs=pl.BlockSpec((1, 1, N), lambda b: (b, 0, 0)),
                out_shape=jax.ShapeDtypeStruct((B, 1, N), F32),
            )(*common)
    s = s.reshape(B, N)

    out = pl.pallas_call(
        _head_body,
        out_shape=jax.ShapeDtypeStruct((B, 40), F32),
    )(s, L1W, vec(g6), vec(be6), L2W, vec(L2b), vec(g7), vec(be7),
      L3W, vec(L3b))
    return out


# speedup vs baseline: 5.1859x; 1.4127x over previous
import functools

import jax
import jax.numpy as jnp
from jax import lax
from jax.experimental import pallas as pl
from jax.experimental.pallas import tpu as pltpu
from jax.experimental.pallas import tpu_sc as plsc

EPS = 1e-5
K = 20
N = 1024
B = 8
NEG = -3e38
F32 = jnp.float32
HIGH = jax.lax.Precision.HIGHEST
NTILE = 16
NCORE = 2
RPT = N // NTILE
BPC = B // NCORE


def _graph_body(x_ref, xf_ref, w1_ref, idx_ref, dinv_ref, sf_ref, hd_ref):
    xr = x_ref[0]
    xc = xf_ref[0]
    sq = xr * xr
    xx_r = (sq[0:1, :] + sq[1:2, :]) + sq[2:3, :]
    sqc = xc * xc
    xx_c = (sqc[:, 0:1] + sqc[:, 1:2]) + sqc[:, 2:3]
    inner = -2.0 * lax.dot_general(xc, xr, (((1,), (0,)), ((), ())),
                                   preferred_element_type=F32)
    work = (-xx_r - inner) - xx_c
    iota_s = lax.broadcasted_iota(jnp.int32, (N, N), 0)
    for r in range(K):
        m = jnp.max(work, axis=0, keepdims=True)
        am = jnp.min(jnp.where(work == m, iota_s, N), axis=0, keepdims=True)
        idx_ref[0, r:r + 1, :] = am
        work = jnp.where(iota_s == am, NEG, work)
    selt = work <= NEG * 0.5
    diag = iota_s == lax.broadcasted_iota(jnp.int32, (N, N), 1)
    awt = jnp.where(diag | selt, 1.0, 0.0).astype(F32)
    deg = jnp.sum(awt, axis=1, keepdims=True)
    dinv = jnp.where(deg > 0, 1.0 / jnp.sqrt(deg), 0.0)
    sf = jnp.sum(jnp.where(diag & selt, 1.0, 0.0), axis=1, keepdims=True)
    dinv_ref[0] = dinv
    sf_ref[0] = sf
    hd_ref[0] = dinv * lax.dot_general(xc, w1_ref[...], (((1,), (0,)), ((), ())),
                                       preferred_element_type=F32)


@functools.lru_cache(maxsize=None)
def _make_sc_agg(d):
    nslice = d // 64
    ntask = B * nslice
    nworker = NCORE * NTILE
    rounds = -(-ntask // nworker)
    half = (N // 2) * 64
    mesh = plsc.VectorSubcoreMesh(core_axis_name="c", subcore_axis_name="s")

    @functools.partial(
        pl.kernel,
        mesh=mesh,
        compiler_params=pltpu.CompilerParams(needs_layout_passes=False),
        out_type=jax.ShapeDtypeStruct((B, nslice, 1, N * 64), F32),
        scratch_types=[
            pltpu.VMEM((K * N,), jnp.int32),
            pltpu.VMEM((half,), F32),
            pltpu.VMEM((N * 64,), F32),
        ],
    )
    def sck(idx_hbm, hd_hbm, zero_hbm, z_hbm, idx_v, sbuf, acc):
        c = lax.axis_index("c")
        s = lax.axis_index("s")
        w = s * NCORE + c
        iota = lax.iota(jnp.int32, 16)
        for t in range(rounds):
            task = w + t * nworker

            @pl.when(task < ntask)
            def _():
                bg = task // nslice
                ci = task % nslice
                pltpu.sync_copy(zero_hbm, acc)
                pltpu.sync_copy(idx_hbm.at[bg, 0], idx_v)
                for hf in range(2):
                    pltpu.sync_copy(hd_hbm.at[bg, ci, hf, 0], sbuf)

                    @plsc.parallel_loop(0, K * 32, unroll=4)
                    def ebody(q):
                        k = q // 32
                        r0 = (q % 32) * 16
                        srcl = r0 + iota
                        dstv = plsc.load_gather(
                            idx_v, [k * N + hf * (N // 2) + srcl])
                        for f in range(64):
                            v = plsc.load_gather(sbuf, [f * (N // 2) + srcl])
                            plsc.addupdate_scatter(acc, [f * N + dstv], v)
                pltpu.sync_copy(acc, z_hbm.at[bg, ci, 0])

    return sck


def _sc_aggregate(d):
    sck = _make_sc_agg(d)

    def run(idxt, hd, zeros64):
        nslice = d // 64
        hd5 = hd.reshape(B, 2, N // 2, nslice, 64).transpose(0, 3, 1, 4, 2)
        hd5 = hd5.reshape(B, nslice, 2, 1, half_flat)
        z5 = sck(idxt.reshape(B, 1, K * N), hd5, zeros64)
        z = z5.reshape(B, nslice, 64, N).transpose(0, 3, 1, 2)
        return z.reshape(B, N, d)

    return run


half_flat = (N // 2) * 64


def _epilogue(z_ref, hd_ref, dinv_ref, sf_ref, b_ref, g_ref, be_ref):
    dinv = dinv_ref[0]
    sf = sf_ref[0]
    inv_sqrt = 1.0 / jnp.sqrt(jnp.array(1.0 + EPS, F32))
    t = dinv * (z_ref[0] + (1.0 - sf) * hd_ref[0]) + b_ref[...]
    t = t * (g_ref[...] * inv_sqrt) + be_ref[...]
    return jnp.where(t >= 0, t, 0.2 * t)


def _layer_body(z_ref, hd_ref, dinv_ref, sf_ref, b_ref, g_ref, be_ref,
                w_ref, out_ref):
    h = _epilogue(z_ref, hd_ref, dinv_ref, sf_ref, b_ref, g_ref, be_ref)
    out_ref[0] = dinv_ref[0] * lax.dot_general(
        h, w_ref[...], (((1,), (0,)), ((), ())), preferred_element_type=F32)


def _pool_body(z_ref, hd_ref, dinv_ref, sf_ref, b_ref, g_ref, be_ref,
               out_ref):
    h = _epilogue(z_ref, hd_ref, dinv_ref, sf_ref, b_ref, g_ref, be_ref)
    out_ref[0] = jnp.sum(h, axis=0, keepdims=True)


def _head_body(s_ref, l1_ref, g6_ref, e6_ref, l2_ref, l2b_ref,
               g7_ref, e7_ref, l3_ref, l3b_ref, out_ref):
    s = s_ref[...]
    inv_sqrt = 1.0 / jnp.sqrt(jnp.array(1.0 + EPS, F32))
    w1a = l1_ref[0:N, :]
    w1b = l1_ref[N:2 * N, :]
    y = (lax.dot_general(s * (1.0 / N), w1a, (((1,), (0,)), ((), ())),
                         preferred_element_type=F32)
         + lax.dot_general(s, w1b, (((1,), (0,)), ((), ())),
                           preferred_element_type=F32))
    y = y * (g6_ref[...] * inv_sqrt) + e6_ref[...]
    y = jnp.where(y >= 0, y, 0.2 * y)
    y = lax.dot_general(y, l2_ref[...], (((1,), (0,)), ((), ())),
                        preferred_element_type=F32)
    y = (y + l2b_ref[...]) * (g7_ref[...] * inv_sqrt) + e7_ref[...]
    y = jnp.where(y >= 0, y, 0.2 * y)
    out_ref[...] = (lax.dot_general(y, l3_ref[...], (((1,), (0,)), ((), ())),
                                    preferred_element_type=F32)
                    + l3b_ref[...])


def _full(shape):
    return pl.BlockSpec(shape, lambda b: (0,) * len(shape))


def _bspec(shape):
    return pl.BlockSpec((1,) + shape, lambda b: (b,) + (0,) * len(shape))


def kernel(x, W1, b1, W2, b2, W3, b3, W4, b4, W5, b5,
           g1, be1, g2, be2, g3, be3, g4, be4, g5, be5, g6, be6, g7, be7,
           L1W, L2W, L2b, L3W, L3b):
    xf = jnp.transpose(x, (0, 2, 1))
    vec = lambda v: v.reshape(1, -1)

    idxt, dinv, sf, hd = pl.pallas_call(
        _graph_body,
        grid=(B,),
        in_specs=[_bspec((3, N)), _bspec((N, 3)), _full(W1.shape)],
        out_specs=[_bspec((K, N)), _bspec((N, 1)), _bspec((N, 1)),
                   _bspec((N, W1.shape[1]))],
        out_shape=[jax.ShapeDtypeStruct((B, K, N), jnp.int32),
                   jax.ShapeDtypeStruct((B, N, 1), F32),
                   jax.ShapeDtypeStruct((B, N, 1), F32),
                   jax.ShapeDtypeStruct((B, N, W1.shape[1]), F32)],
    )(x, xf, W1)

    layer_params = ((b1, g1, be1, W2), (b2, g2, be2, W3),
                    (b3, g3, be3, W4), (b4, g4, be4, W5),
                    (b5, g5, be5, None))
    zeros64 = jnp.zeros((N * 64,), F32)
    for bb, g, be, wn in layer_params:
        d = hd.shape[-1]
        z = _sc_aggregate(d)(idxt, hd, zeros64)
        common = (z, hd, dinv, sf, vec(bb), vec(g), vec(be))
        common_specs = [_bspec((N, d)), _bspec((N, d)), _bspec((N, 1)),
                        _bspec((N, 1)), _full((1, d)), _full((1, d)),
                        _full((1, d))]
        if wn is not None:
            hd = pl.pallas_call(
                _layer_body,
                grid=(B,),
                in_specs=common_specs + [_full(wn.shape)],
                out_specs=_bspec((N, wn.shape[1])),
                out_shape=jax.ShapeDtypeStruct((B, N, wn.shape[1]), F32),
            )(*common, wn)
        else:
            s = pl.pallas_call(
                _pool_body,
                grid=(B,),
                in_specs=common_specs,
                out_specs=pl.BlockSpec((1, 1, N), lambda b: (b, 0, 0)),
                out_shape=jax.ShapeDtypeStruct((B, 1, N), F32),
            )(*common)
    s = s.reshape(B, N)

    out = pl.pallas_call(
        _head_body,
        out_shape=jax.ShapeDtypeStruct((B, 40), F32),
    )(s, L1W, vec(g6), vec(be6), L2W, vec(L2b), vec(g7), vec(be7),
      L3W, vec(L3b))
    return out
